# baseline (speedup 1.0000x reference)
import functools

import jax
import jax.numpy as jnp
from jax import lax
from jax.experimental import pallas as pl
from jax.experimental.pallas import tpu as pltpu
from jax.experimental.pallas import tpu_sc as plsc

VOCAB = 32000
D_MODEL = 1024
N_EXPERTS = 8
RANK = 32
T = 2048
ER = N_EXPERTS * RANK


def _make_sc_gather():
    info = plsc.get_sparse_core_info()
    nc, ns = info.num_cores, info.num_subcores
    nw = nc * ns
    b_per_w = T // nw

    mesh = plsc.VectorSubcoreMesh(core_axis_name="c", subcore_axis_name="s")

    @functools.partial(
        pl.kernel,
        mesh=mesh,
        out_type=jax.ShapeDtypeStruct((T, D_MODEL), jnp.float32),
        scratch_types=[
            pltpu.VMEM((b_per_w,), jnp.int32),
            pltpu.VMEM((b_per_w, D_MODEL), jnp.float32),
            pltpu.SemaphoreType.DMA,
        ],
    )
    def gather_kernel(idx_hbm, table_hbm, out_hbm, idx_v, rows_v, sem):
        wid = lax.axis_index("s") * nc + lax.axis_index("c")
        base = wid * b_per_w
        pltpu.sync_copy(idx_hbm.at[pl.ds(base, b_per_w)], idx_v)
        pltpu.async_copy(table_hbm.at[idx_v], rows_v, sem).wait()
        pltpu.sync_copy(rows_v, out_hbm.at[pl.ds(base, b_per_w)])

    return gather_kernel


_sc_gather = None


def _gather_rows(idx_flat, emb):
    global _sc_gather
    if _sc_gather is None:
        _sc_gather = _make_sc_gather()
    return _sc_gather(idx_flat, emb)


TC = 256
NCHUNK = T // TC
VB = 1280
NVB = VOCAB // VB


def _route_ln_chunk(x_ref, a2_ref, b2_ref, g_ref, bta_ref):
    x = x_ref[...]
    proj = jnp.dot(x, a2_ref[...], preferred_element_type=jnp.float32)
    col = lax.broadcasted_iota(jnp.int32, (ER, N_EXPERTS), 0) // RANK
    sel_mat = (col == lax.broadcasted_iota(jnp.int32, (ER, N_EXPERTS), 1)
               ).astype(jnp.float32)
    p2 = proj * proj
    fid = jnp.dot(p2, sel_mat, preferred_element_type=jnp.float32)
    m = jnp.max(fid, axis=1, keepdims=True)
    eidx = lax.broadcasted_iota(jnp.int32, (TC, N_EXPERTS), 1)
    win = jnp.min(jnp.where(fid >= m, eidx, N_EXPERTS), axis=1, keepdims=True)
    colexp = lax.broadcasted_iota(jnp.int32, (TC, ER), 1) // RANK
    pm = jnp.where(colexp == win, proj, 0.0)
    delta = jnp.dot(pm, b2_ref[...], preferred_element_type=jnp.float32)
    xs = x + delta
    mu = jnp.mean(xs, axis=1, keepdims=True)
    var = jnp.mean((xs - mu) * (xs - mu), axis=1, keepdims=True)
    xn = (xs - mu) * lax.rsqrt(var + 1e-5)
    return (xn * g_ref[...] + bta_ref[...]).astype(jnp.bfloat16)


def _fused_kernel(x_ref, a2_ref, b2_ref, g_ref, bta_ref, emb_ref, out_ref,
                  xn_scr):
    i = pl.program_id(0)

    @pl.when(i < NCHUNK)
    def _():
        xn_scr[pl.ds(i * TC, TC), :] = _route_ln_chunk(
            x_ref, a2_ref, b2_ref, g_ref, bta_ref)

    @pl.when(i >= NCHUNK)
    def _():
        out_ref[...] = lax.dot_general(
            xn_scr[...], emb_ref[...].astype(jnp.bfloat16),
            dimension_numbers=(((1,), (1,)), ((), ())),
            preferred_element_type=jnp.float32,
        )


def _fused(x, a2, b2, gamma2, beta2, emb):
    return pl.pallas_call(
        _fused_kernel,
        grid=(NCHUNK + NVB,),
        in_specs=[
            pl.BlockSpec((TC, D_MODEL),
                         lambda i: (jnp.minimum(i, NCHUNK - 1), 0)),
            pl.BlockSpec((D_MODEL, ER), lambda i: (0, 0)),
            pl.BlockSpec((ER, D_MODEL), lambda i: (0, 0)),
            pl.BlockSpec((1, D_MODEL), lambda i: (0, 0)),
            pl.BlockSpec((1, D_MODEL), lambda i: (0, 0)),
            pl.BlockSpec((VB, D_MODEL),
                         lambda i: (jnp.maximum(i - NCHUNK, 0), 0)),
        ],
        out_specs=pl.BlockSpec((T, VB),
                               lambda i: (0, jnp.maximum(i - NCHUNK, 0))),
        out_shape=jax.ShapeDtypeStruct((T, VOCAB), jnp.float32),
        scratch_shapes=[pltpu.VMEM((T, D_MODEL), jnp.bfloat16)],
        compiler_params=pltpu.CompilerParams(
            dimension_semantics=("arbitrary",),
        ),
    )(x, a2, b2, gamma2, beta2, emb)


def kernel(idx, emb, A, Bm, gamma, beta):
    idx_flat = idx.reshape(T).astype(jnp.int32)
    x = _gather_rows(idx_flat, emb)
    a2 = jnp.transpose(A, (1, 0, 2)).reshape(D_MODEL, ER)
    b2 = Bm.reshape(ER, D_MODEL)
    logits = _fused(x, a2, b2,
                    gamma.reshape(1, D_MODEL), beta.reshape(1, D_MODEL),
                    emb)
    return logits.reshape(1, T, VOCAB)


# speedup vs baseline: 1.0116x; 1.0116x over previous
import functools

import jax
import jax.numpy as jnp
from jax import lax
from jax.experimental import pallas as pl
from jax.experimental.pallas import tpu as pltpu
from jax.experimental.pallas import tpu_sc as plsc

VOCAB = 32000
D_MODEL = 1024
N_EXPERTS = 8
RANK = 32
T = 2048
ER = N_EXPERTS * RANK


def _make_sc_gather():
    info = plsc.get_sparse_core_info()
    nc, ns = info.num_cores, info.num_subcores
    nw = nc * ns
    b_per_w = T // nw

    mesh = plsc.VectorSubcoreMesh(core_axis_name="c", subcore_axis_name="s")

    @functools.partial(
        pl.kernel,
        mesh=mesh,
        out_type=jax.ShapeDtypeStruct((T, D_MODEL), jnp.float32),
        scratch_types=[
            pltpu.VMEM((b_per_w,), jnp.int32),
            pltpu.VMEM((b_per_w, D_MODEL), jnp.float32),
            pltpu.SemaphoreType.DMA,
        ],
    )
    def gather_kernel(idx_hbm, table_hbm, out_hbm, idx_v, rows_v, sem):
        wid = lax.axis_index("s") * nc + lax.axis_index("c")
        base = wid * b_per_w
        pltpu.sync_copy(idx_hbm.at[pl.ds(base, b_per_w)], idx_v)
        pltpu.async_copy(table_hbm.at[idx_v], rows_v, sem).wait()
        pltpu.sync_copy(rows_v, out_hbm.at[pl.ds(base, b_per_w)])

    return gather_kernel


_sc_gather = None


def _gather_rows(idx_flat, emb):
    global _sc_gather
    if _sc_gather is None:
        _sc_gather = _make_sc_gather()
    return _sc_gather(idx_flat, emb)


TC = 512
NCHUNK = T // TC
VB = 1280
NVB = VOCAB // VB


def _route_ln_chunk(x_ref, a2_ref, b2_ref, g_ref, bta_ref):
    x = x_ref[...]
    proj = jnp.dot(x, a2_ref[...], preferred_element_type=jnp.float32)
    col = lax.broadcasted_iota(jnp.int32, (ER, N_EXPERTS), 0) // RANK
    sel_mat = (col == lax.broadcasted_iota(jnp.int32, (ER, N_EXPERTS), 1)
               ).astype(jnp.float32)
    p2 = proj * proj
    fid = jnp.dot(p2, sel_mat, preferred_element_type=jnp.float32)
    m = jnp.max(fid, axis=1, keepdims=True)
    eidx = lax.broadcasted_iota(jnp.int32, (TC, N_EXPERTS), 1)
    win = jnp.min(jnp.where(fid >= m, eidx, N_EXPERTS), axis=1, keepdims=True)
    colexp = lax.broadcasted_iota(jnp.int32, (TC, ER), 1) // RANK
    pm = jnp.where(colexp == win, proj, 0.0)
    delta = jnp.dot(pm, b2_ref[...], preferred_element_type=jnp.float32)
    xs = x + delta
    mu = jnp.mean(xs, axis=1, keepdims=True)
    var = jnp.mean((xs - mu) * (xs - mu), axis=1, keepdims=True)
    xn = (xs - mu) * lax.rsqrt(var + 1e-5)
    return (xn * g_ref[...] + bta_ref[...]).astype(jnp.bfloat16)


def _fused_kernel(x_ref, a2_ref, b2_ref, g_ref, bta_ref, emb_ref, out_ref,
                  xn_scr):
    i = pl.program_id(0)

    @pl.when(i < NCHUNK)
    def _():
        xn_scr[pl.ds(i * TC, TC), :] = _route_ln_chunk(
            x_ref, a2_ref, b2_ref, g_ref, bta_ref)

    @pl.when(i >= NCHUNK)
    def _():
        out_ref[...] = lax.dot_general(
            xn_scr[...], emb_ref[...].astype(jnp.bfloat16),
            dimension_numbers=(((1,), (1,)), ((), ())),
            preferred_element_type=jnp.float32,
        )


def _fused(x, a2, b2, gamma2, beta2, emb):
    return pl.pallas_call(
        _fused_kernel,
        grid=(NCHUNK + NVB,),
        in_specs=[
            pl.BlockSpec((TC, D_MODEL),
                         lambda i: (jnp.minimum(i, NCHUNK - 1), 0)),
            pl.BlockSpec((D_MODEL, ER), lambda i: (0, 0)),
            pl.BlockSpec((ER, D_MODEL), lambda i: (0, 0)),
            pl.BlockSpec((1, D_MODEL), lambda i: (0, 0)),
            pl.BlockSpec((1, D_MODEL), lambda i: (0, 0)),
            pl.BlockSpec((VB, D_MODEL),
                         lambda i: (jnp.maximum(i - NCHUNK, 0), 0)),
        ],
        out_specs=pl.BlockSpec((T, VB),
                               lambda i: (0, jnp.maximum(i - NCHUNK, 0))),
        out_shape=jax.ShapeDtypeStruct((T, VOCAB), jnp.float32),
        scratch_shapes=[pltpu.VMEM((T, D_MODEL), jnp.bfloat16)],
        compiler_params=pltpu.CompilerParams(
            dimension_semantics=("arbitrary",),
        ),
    )(x, a2, b2, gamma2, beta2, emb)


def kernel(idx, emb, A, Bm, gamma, beta):
    idx_flat = idx.reshape(T).astype(jnp.int32)
    x = _gather_rows(idx_flat, emb)
    a2 = jnp.transpose(A, (1, 0, 2)).reshape(D_MODEL, ER)
    b2 = Bm.reshape(ER, D_MODEL)
    logits = _fused(x, a2, b2,
                    gamma.reshape(1, D_MODEL), beta.reshape(1, D_MODEL),
                    emb)
    return logits.reshape(1, T, VOCAB)


# speedup vs baseline: 1.0124x; 1.0008x over previous
import functools

import jax
import jax.numpy as jnp
from jax import lax
from jax.experimental import pallas as pl
from jax.experimental.pallas import tpu as pltpu
from jax.experimental.pallas import tpu_sc as plsc

VOCAB = 32000
D_MODEL = 1024
N_EXPERTS = 8
RANK = 32
T = 2048
ER = N_EXPERTS * RANK


def _make_sc_gather():
    info = plsc.get_sparse_core_info()
    nc, ns = info.num_cores, info.num_subcores
    nw = nc * ns
    b_per_w = T // nw

    mesh = plsc.VectorSubcoreMesh(core_axis_name="c", subcore_axis_name="s")

    @functools.partial(
        pl.kernel,
        mesh=mesh,
        out_type=jax.ShapeDtypeStruct((T, D_MODEL), jnp.float32),
        scratch_types=[
            pltpu.VMEM((b_per_w,), jnp.int32),
            pltpu.VMEM((b_per_w, D_MODEL), jnp.float32),
            pltpu.SemaphoreType.DMA,
            pltpu.SemaphoreType.DMA,
        ],
    )
    def gather_kernel(idx_hbm, table_hbm, out_hbm, idx_v, rows_v, sem, osem):
        wid = lax.axis_index("s") * nc + lax.axis_index("c")
        base = wid * b_per_w
        half = b_per_w // 2
        pltpu.sync_copy(idx_hbm.at[pl.ds(base, b_per_w)], idx_v)
        pltpu.async_copy(table_hbm.at[idx_v.at[pl.ds(0, half)]],
                         rows_v.at[pl.ds(0, half)], sem).wait()
        out0 = pltpu.make_async_copy(rows_v.at[pl.ds(0, half)],
                                     out_hbm.at[pl.ds(base, half)], osem)
        out0.start()
        pltpu.async_copy(table_hbm.at[idx_v.at[pl.ds(half, half)]],
                         rows_v.at[pl.ds(half, half)], sem).wait()
        out0.wait()
        pltpu.sync_copy(rows_v.at[pl.ds(half, half)],
                        out_hbm.at[pl.ds(base + half, half)])

    return gather_kernel


_sc_gather = None


def _gather_rows(idx_flat, emb):
    global _sc_gather
    if _sc_gather is None:
        _sc_gather = _make_sc_gather()
    return _sc_gather(idx_flat, emb)


TC = 1024
NCHUNK = T // TC
VB = 1280
NVB = VOCAB // VB


def _route_ln_chunk(x_ref, a2_ref, b2_ref, g_ref, bta_ref):
    x = x_ref[...]
    proj = jnp.dot(x, a2_ref[...], preferred_element_type=jnp.float32)
    col = lax.broadcasted_iota(jnp.int32, (ER, N_EXPERTS), 0) // RANK
    sel_mat = (col == lax.broadcasted_iota(jnp.int32, (ER, N_EXPERTS), 1)
               ).astype(jnp.float32)
    p2 = proj * proj
    fid = jnp.dot(p2, sel_mat, preferred_element_type=jnp.float32)
    m = jnp.max(fid, axis=1, keepdims=True)
    eidx = lax.broadcasted_iota(jnp.int32, (TC, N_EXPERTS), 1)
    win = jnp.min(jnp.where(fid >= m, eidx, N_EXPERTS), axis=1, keepdims=True)
    colexp = lax.broadcasted_iota(jnp.int32, (TC, ER), 1) // RANK
    pm = jnp.where(colexp == win, proj, 0.0)
    delta = jnp.dot(pm, b2_ref[...], preferred_element_type=jnp.float32)
    xs = x + delta
    mu = jnp.mean(xs, axis=1, keepdims=True)
    var = jnp.mean((xs - mu) * (xs - mu), axis=1, keepdims=True)
    xn = (xs - mu) * lax.rsqrt(var + 1e-5)
    return (xn * g_ref[...] + bta_ref[...]).astype(jnp.bfloat16)


def _fused_kernel(x_ref, a2_ref, b2_ref, g_ref, bta_ref, emb_ref, out_ref,
                  xn_scr):
    i = pl.program_id(0)

    @pl.when(i < NCHUNK)
    def _():
        xn_scr[pl.ds(i * TC, TC), :] = _route_ln_chunk(
            x_ref, a2_ref, b2_ref, g_ref, bta_ref)

    @pl.when(i >= NCHUNK)
    def _():
        out_ref[...] = lax.dot_general(
            xn_scr[...], emb_ref[...].astype(jnp.bfloat16),
            dimension_numbers=(((1,), (1,)), ((), ())),
            preferred_element_type=jnp.float32,
        )


def _fused(x, a2, b2, gamma2, beta2, emb):
    return pl.pallas_call(
        _fused_kernel,
        grid=(NCHUNK + NVB,),
        in_specs=[
            pl.BlockSpec((TC, D_MODEL),
                         lambda i: (jnp.minimum(i, NCHUNK - 1), 0)),
            pl.BlockSpec((D_MODEL, ER), lambda i: (0, 0)),
            pl.BlockSpec((ER, D_MODEL), lambda i: (0, 0)),
            pl.BlockSpec((1, D_MODEL), lambda i: (0, 0)),
            pl.BlockSpec((1, D_MODEL), lambda i: (0, 0)),
            pl.BlockSpec((VB, D_MODEL),
                         lambda i: (jnp.maximum(i - NCHUNK, 0), 0)),
        ],
        out_specs=pl.BlockSpec((T, VB),
                               lambda i: (0, jnp.maximum(i - NCHUNK, 0))),
        out_shape=jax.ShapeDtypeStruct((T, VOCAB), jnp.float32),
        scratch_shapes=[pltpu.VMEM((T, D_MODEL), jnp.bfloat16)],
        compiler_params=pltpu.CompilerParams(
            dimension_semantics=("arbitrary",),
        ),
    )(x, a2, b2, gamma2, beta2, emb)


def kernel(idx, emb, A, Bm, gamma, beta):
    idx_flat = idx.reshape(T).astype(jnp.int32)
    x = _gather_rows(idx_flat, emb)
    a2 = jnp.transpose(A, (1, 0, 2)).reshape(D_MODEL, ER)
    b2 = Bm.reshape(ER, D_MODEL)
    logits = _fused(x, a2, b2,
                    gamma.reshape(1, D_MODEL), beta.reshape(1, D_MODEL),
                    emb)
    return logits.reshape(1, T, VOCAB)
